# baseline (speedup 1.0000x reference)
import functools

import jax
import jax.numpy as jnp
from jax import lax
from jax.experimental import pallas as pl
from jax.experimental.pallas import tpu as pltpu
from jax.experimental.pallas import tpu_sc as plsc

Q_EXP = 0.7
B_ROWS = 16384
D_COLS = 1000
COLS_PER_BLOCK = 512


def _make_sc_gather(num_rows: int, table_rows: int):
    info = plsc.get_sparse_core_info()
    nc, ns = info.num_cores, info.num_subcores
    nw = nc * ns
    chunk = num_rows // nw
    mesh = plsc.VectorSubcoreMesh(core_axis_name="c", subcore_axis_name="s")

    @functools.partial(
        pl.kernel,
        mesh=mesh,
        out_type=jax.ShapeDtypeStruct((num_rows,), jnp.float32),
        scratch_types=[
            pltpu.VMEM((chunk,), jnp.int32),
            pltpu.VMEM((chunk,), jnp.float32),
            pltpu.SemaphoreType.DMA,
        ],
    )
    def gather_k(ids_hbm, p_hbm, w_hbm, idx_v, vals_v, sem):
        wid = lax.axis_index("s") * nc + lax.axis_index("c")
        base = wid * chunk
        pltpu.sync_copy(ids_hbm.at[pl.ds(base, chunk)], idx_v)
        pltpu.async_copy(p_hbm.at[idx_v], vals_v, sem).wait()
        pltpu.sync_copy(vals_v, w_hbm.at[pl.ds(base, chunk)])

    return gather_k


def _tc_loss_body(yp_ref, yt_ref, l_ref):
    yp = yp_ref[...]
    yt = yt_ref[...]
    t = jnp.sum(yp * yt, axis=0, keepdims=True)
    s = jnp.sum(jnp.exp(yp - t), axis=0, keepdims=True)
    l_ref[...] = (1.0 - (1.0 / s) ** Q_EXP) / Q_EXP


def _tc_loss(yp_t, yt_t):
    nb = B_ROWS // COLS_PER_BLOCK
    return pl.pallas_call(
        _tc_loss_body,
        grid=(nb,),
        in_specs=[
            pl.BlockSpec((D_COLS, COLS_PER_BLOCK), lambda i: (0, i)),
            pl.BlockSpec((D_COLS, COLS_PER_BLOCK), lambda i: (0, i)),
        ],
        out_specs=pl.BlockSpec((1, COLS_PER_BLOCK), lambda i: (0, i)),
        out_shape=jax.ShapeDtypeStruct((1, B_ROWS), jnp.float32),
    )(yp_t, yt_t)


def _tc_combine_body(l_ref, w_ref, out_ref):
    out_ref[0, 0] = jnp.sum(l_ref[...] * w_ref[...]) * (1.0 / B_ROWS)


def _tc_combine(l2d, w2d):
    out = pl.pallas_call(
        _tc_combine_body,
        out_specs=pl.BlockSpec(memory_space=pltpu.SMEM),
        out_shape=jax.ShapeDtypeStruct((1, 1), jnp.float32),
    )(l2d, w2d)
    return out[0, 0]


def kernel(y_pred, y_true, ids, p):
    ids32 = ids.astype(jnp.int32)
    p1d = p.reshape(-1)
    gather_k = _make_sc_gather(B_ROWS, p1d.shape[0])
    w = gather_k(ids32, p1d)
    l2d = _tc_loss(y_pred.T, y_true.T)
    return _tc_combine(l2d, w.reshape(1, B_ROWS))


# speedup vs baseline: 1.0966x; 1.0966x over previous
import functools

import jax
import jax.numpy as jnp
from jax import lax
from jax.experimental import pallas as pl
from jax.experimental.pallas import tpu as pltpu
from jax.experimental.pallas import tpu_sc as plsc

Q_EXP = 0.7
B_ROWS = 16384
D_COLS = 1000
COLS_PER_BLOCK = 1024


def _make_sc_gather(num_rows: int, table_rows: int):
    info = plsc.get_sparse_core_info()
    nc, ns = info.num_cores, info.num_subcores
    nw = nc * ns
    chunk = num_rows // nw
    mesh = plsc.VectorSubcoreMesh(core_axis_name="c", subcore_axis_name="s")

    @functools.partial(
        pl.kernel,
        mesh=mesh,
        out_type=jax.ShapeDtypeStruct((num_rows,), jnp.float32),
        scratch_types=[
            pltpu.VMEM((chunk,), jnp.int32),
            pltpu.VMEM((chunk,), jnp.float32),
            pltpu.SemaphoreType.DMA,
        ],
    )
    def gather_k(ids_hbm, p_hbm, w_hbm, idx_v, vals_v, sem):
        wid = lax.axis_index("s") * nc + lax.axis_index("c")
        base = wid * chunk
        pltpu.sync_copy(ids_hbm.at[pl.ds(base, chunk)], idx_v)
        pltpu.async_copy(p_hbm.at[idx_v], vals_v, sem).wait()
        pltpu.sync_copy(vals_v, w_hbm.at[pl.ds(base, chunk)])

    return gather_k


def _tc_loss_body(yp_ref, yt_ref, l_ref):
    yp = yp_ref[...]
    yt = yt_ref[...]
    t = jnp.sum(yp * yt, axis=0, keepdims=True)
    s = jnp.sum(jnp.exp(yp - t), axis=0, keepdims=True)
    l_ref[...] = (1.0 - (1.0 / s) ** Q_EXP) / Q_EXP


def _tc_loss(yp_t, yt_t):
    nb = B_ROWS // COLS_PER_BLOCK
    return pl.pallas_call(
        _tc_loss_body,
        grid=(nb,),
        in_specs=[
            pl.BlockSpec((D_COLS, COLS_PER_BLOCK), lambda i: (0, i)),
            pl.BlockSpec((D_COLS, COLS_PER_BLOCK), lambda i: (0, i)),
        ],
        out_specs=pl.BlockSpec((1, COLS_PER_BLOCK), lambda i: (0, i)),
        out_shape=jax.ShapeDtypeStruct((1, B_ROWS), jnp.float32),
    )(yp_t, yt_t)


def _tc_combine_body(l_ref, w_ref, out_ref):
    out_ref[0, 0] = jnp.sum(l_ref[...] * w_ref[...]) * (1.0 / B_ROWS)


def _tc_combine(l2d, w2d):
    out = pl.pallas_call(
        _tc_combine_body,
        out_specs=pl.BlockSpec(memory_space=pltpu.SMEM),
        out_shape=jax.ShapeDtypeStruct((1, 1), jnp.float32),
    )(l2d, w2d)
    return out[0, 0]


def kernel(y_pred, y_true, ids, p):
    ids32 = ids.astype(jnp.int32)
    p1d = p.reshape(-1)
    gather_k = _make_sc_gather(B_ROWS, p1d.shape[0])
    w = gather_k(ids32, p1d)
    l2d = _tc_loss(y_pred.T, y_true.T)
    return _tc_combine(l2d, w.reshape(1, B_ROWS))
